# baseline (speedup 1.0000x reference)
import functools

import jax
import jax.numpy as jnp
from jax import lax
from jax.experimental import pallas as pl
from jax.experimental.pallas import tpu as pltpu
from jax.experimental.pallas import tpu_sc as plsc

_NC = 2
_NS = 16
_NW = _NC * _NS
_C = 128
_NBUF = 5
_RB = 1000


def _make_deg_kernel(NP, KC):
    TR = NP // _NS
    mesh = plsc.VectorSubcoreMesh(core_axis_name="c", subcore_axis_name="s")

    @functools.partial(
        pl.kernel,
        out_type=jax.ShapeDtypeStruct((_NC * NP,), jnp.float32),
        mesh=mesh,
        scratch_types=[
            pltpu.VMEM((KC, _C), jnp.int32),
            pltpu.VMEM((_C,), jnp.float32),
            pltpu.VMEM((TR,), jnp.float32),
            pltpu.VMEM_SHARED((NP,), jnp.float32),
            pltpu.SemaphoreType.DMA,
        ],
        compiler_params=pltpu.CompilerParams(use_tc_tiling_on_sc=False),
    )
    def deg_kernel(dst_hbm, ones_hbm, zrow_hbm, out_hbm, dst_v, ones_v, buf_v,
                   acc_sh, ssem):
        c = lax.axis_index("c")
        s = lax.axis_index("s")
        wid = s * _NC + c
        pltpu.sync_copy(zrow_hbm, buf_v)
        pltpu.sync_copy(buf_v, acc_sh.at[pl.ds(s * TR, TR)])
        pltpu.sync_copy(ones_hbm, ones_v)
        pltpu.sync_copy(dst_hbm.at[wid], dst_v)
        plsc.subcore_barrier()

        def fire(j, carry):
            pltpu.async_copy(ones_v, acc_sh.at[dst_v.at[j]], ssem, add=True)
            return carry

        lax.fori_loop(0, KC, fire, 0)

        def drain(j, carry):
            pltpu.make_async_copy(ones_v, acc_sh.at[dst_v.at[0]], ssem).wait()
            return carry

        lax.fori_loop(0, KC, drain, 0)
        plsc.subcore_barrier()
        pltpu.sync_copy(acc_sh.at[pl.ds(s * TR, TR)], buf_v)
        pltpu.sync_copy(buf_v, out_hbm.at[pl.ds(c * NP + s * TR, TR)])

    return deg_kernel


def _make_prop_kernel(NP, KC, H):
    TR = NP // _NS
    G = KC // _NBUF
    RING = _NBUF
    mesh = plsc.VectorSubcoreMesh(core_axis_name="c", subcore_axis_name="s")

    @functools.partial(
        pl.kernel,
        out_type=jax.ShapeDtypeStruct((_NC, NP, H), jnp.float32),
        mesh=mesh,
        scratch_types=[
            pltpu.VMEM((KC, _C), jnp.int32),
            pltpu.VMEM((KC, _C), jnp.int32),
            pltpu.VMEM((RING, _C, H), jnp.float32),
            pltpu.VMEM((TR, H), jnp.float32),
            pltpu.VMEM_SHARED((NP, H), jnp.float32),
            pltpu.SemaphoreType.DMA((RING,)),
            pltpu.SemaphoreType.DMA((RING,)),
        ],
        compiler_params=pltpu.CompilerParams(use_tc_tiling_on_sc=False),
    )
    def prop_kernel(tbl_hbm, src_hbm, dst_hbm, ztile_hbm, out_hbm,
                    src_v, dst_v, rows_v, buf_v, acc_sh, gsem, ssem):
        c = lax.axis_index("c")
        s = lax.axis_index("s")
        wid = s * _NC + c
        pltpu.sync_copy(ztile_hbm, buf_v)
        pltpu.sync_copy(buf_v, acc_sh.at[pl.ds(s * TR, TR)])
        pltpu.sync_copy(src_hbm.at[wid], src_v)
        pltpu.sync_copy(dst_hbm.at[wid], dst_v)
        plsc.subcore_barrier()

        for b in range(_NBUF):
            pltpu.async_copy(tbl_hbm.at[src_v.at[b]], rows_v.at[b], gsem.at[b])

        def body(g, carry):
            for b in range(_NBUF):
                j = g * _NBUF + b
                pltpu.make_async_copy(
                    tbl_hbm.at[src_v.at[j]], rows_v.at[b], gsem.at[b]).wait()
                pltpu.async_copy(
                    rows_v.at[b], acc_sh.at[dst_v.at[j]], ssem.at[b], add=True)

                @pl.when(g < G - 1)
                def _():
                    pltpu.make_async_copy(
                        rows_v.at[b], acc_sh.at[dst_v.at[j]], ssem.at[b]).wait()
                    pltpu.async_copy(
                        tbl_hbm.at[src_v.at[j + _NBUF]], rows_v.at[b], gsem.at[b])
            return carry

        lax.fori_loop(0, G, body, 0)
        for b in range(_NBUF):
            j = KC - _NBUF + b
            pltpu.make_async_copy(
                rows_v.at[b], acc_sh.at[dst_v.at[j]], ssem.at[b]).wait()
        plsc.subcore_barrier()
        pltpu.sync_copy(acc_sh.at[pl.ds(s * TR, TR)], buf_v)
        pltpu.sync_copy(buf_v, out_hbm.at[c, pl.ds(s * TR, TR)])

    return prop_kernel


def _mm1_body(x_ref, w1_ref, dega_ref, degb_ref, dinv_ref, p1_ref):
    d = w1_ref.shape[0]
    dinv = lax.rsqrt(dega_ref[...] + degb_ref[...] + 1.0)
    dinv_ref[...] = dinv
    h = x_ref[:, :d] * x_ref[:, d:d + 1]
    p1_ref[...] = jnp.dot(h, w1_ref[...], preferred_element_type=jnp.float32,
                          precision=lax.Precision.HIGHEST) * dinv


def _layer_body(parts_ref, p1_ref, dinv_ref, b1_ref, w2_ref, q_ref):
    agg = (parts_ref[0] + parts_ref[1] + p1_ref[...]) * dinv_ref[...]
    h1 = jnp.maximum(agg + b1_ref[...], 0.0)
    q_ref[...] = jnp.dot(h1, w2_ref[...], preferred_element_type=jnp.float32,
                         precision=lax.Precision.HIGHEST) * dinv_ref[...]


def _final_body(parts_ref, q_ref, dinv_ref, b2_ref, wd_ref, bd_ref, out_ref):
    i = pl.program_id(0)
    agg = (parts_ref[0] + parts_ref[1] + q_ref[...]) * dinv_ref[...]
    h2 = jnp.maximum(agg + b2_ref[...], 0.0)
    pooled = jnp.sum(h2, axis=0, keepdims=True)
    val = jnp.dot(pooled, wd_ref[...], preferred_element_type=jnp.float32,
                  precision=lax.Precision.HIGHEST)

    @pl.when(i == 0)
    def _():
        out_ref[...] = bd_ref[...]

    out_ref[...] += val


def kernel(x, edge_index, W1, b1, W2, b2, Wd, bd):
    n = x.shape[0]
    d = W1.shape[0]
    H = W1.shape[1]
    e = edge_index.shape[1]

    NP = -(-n // 128) * 128
    EPT = -(-e // _NW)
    KC = -(-(-(-EPT // _C)) // _NBUF) * _NBUF
    EPTP = KC * _C
    NG = n // _RB

    ei = edge_index.astype(jnp.int32)
    if _NW * EPT != e:
        pad = jnp.concatenate([jnp.zeros((1, _NW * EPT - e), jnp.int32),
                               jnp.full((1, _NW * EPT - e), n, jnp.int32)])
        ei = jnp.concatenate([ei, pad], axis=1)
    ei3 = ei.reshape(2, _NW, EPT)
    if EPTP != EPT:
        padt = jnp.concatenate([jnp.zeros((1, _NW, EPTP - EPT), jnp.int32),
                                jnp.full((1, _NW, EPTP - EPT), n, jnp.int32)])
        ei3 = jnp.concatenate([ei3, padt], axis=2)
    src3 = ei3[0].reshape(_NW, KC, _C)
    dst3 = ei3[1].reshape(_NW, KC, _C)

    ones_c = jnp.ones((_C,), jnp.float32)
    zrow = jnp.zeros((NP // _NS,), jnp.float32)
    ztile = jnp.zeros((NP // _NS, H), jnp.float32)

    deg_kernel = _make_deg_kernel(NP, KC)
    prop_kernel = _make_prop_kernel(NP, KC, H)

    deg1d = deg_kernel(dst3, ones_c, zrow)
    dega = deg1d[:NP].reshape(NP, 1)
    degb = deg1d[NP:].reshape(NP, 1)

    dinv, p1 = pl.pallas_call(
        _mm1_body,
        grid=(NG,),
        in_specs=[pl.BlockSpec((_RB, x.shape[1]), lambda i: (i, 0)),
                  pl.BlockSpec((d, H), lambda i: (0, 0)),
                  pl.BlockSpec((_RB, 1), lambda i: (i, 0)),
                  pl.BlockSpec((_RB, 1), lambda i: (i, 0))],
        out_specs=(pl.BlockSpec((_RB, 1), lambda i: (i, 0)),
                   pl.BlockSpec((_RB, H), lambda i: (i, 0))),
        out_shape=(jax.ShapeDtypeStruct((NP, 1), jnp.float32),
                   jax.ShapeDtypeStruct((NP, H), jnp.float32)),
    )(x, W1, dega, degb)

    parts1 = prop_kernel(p1, src3, dst3, ztile)

    q = pl.pallas_call(
        _layer_body,
        grid=(NG,),
        in_specs=[pl.BlockSpec((2, _RB, H), lambda i: (0, i, 0)),
                  pl.BlockSpec((_RB, H), lambda i: (i, 0)),
                  pl.BlockSpec((_RB, 1), lambda i: (i, 0)),
                  pl.BlockSpec((1, H), lambda i: (0, 0)),
                  pl.BlockSpec((H, H), lambda i: (0, 0))],
        out_specs=pl.BlockSpec((_RB, H), lambda i: (i, 0)),
        out_shape=jax.ShapeDtypeStruct((NP, H), jnp.float32),
    )(parts1, p1, dinv, b1.reshape(1, H), W2)

    parts2 = prop_kernel(q, src3, dst3, ztile)

    out = pl.pallas_call(
        _final_body,
        grid=(NG,),
        in_specs=[pl.BlockSpec((2, _RB, H), lambda i: (0, i, 0)),
                  pl.BlockSpec((_RB, H), lambda i: (i, 0)),
                  pl.BlockSpec((_RB, 1), lambda i: (i, 0)),
                  pl.BlockSpec((1, H), lambda i: (0, 0)),
                  pl.BlockSpec((H, 1), lambda i: (0, 0)),
                  pl.BlockSpec((1, 1), lambda i: (0, 0))],
        out_specs=pl.BlockSpec((1, 1), lambda i: (0, 0)),
        out_shape=jax.ShapeDtypeStruct((1, 1), jnp.float32),
    )(parts2, q, dinv, b2.reshape(1, H), Wd, bd.reshape(1, 1))

    return out


# speedup vs baseline: 1.0081x; 1.0081x over previous
import functools

import jax
import jax.numpy as jnp
from jax import lax
from jax.experimental import pallas as pl
from jax.experimental.pallas import tpu as pltpu
from jax.experimental.pallas import tpu_sc as plsc

_NC = 2
_NS = 16
_NW = _NC * _NS
_C = 128
_NBUF = 5
_RB = 1000


def _make_deg_kernel(NP, KC):
    TR = NP // _NS
    mesh = plsc.VectorSubcoreMesh(core_axis_name="c", subcore_axis_name="s")

    @functools.partial(
        pl.kernel,
        out_type=jax.ShapeDtypeStruct((_NC * NP,), jnp.float32),
        mesh=mesh,
        scratch_types=[
            pltpu.VMEM((KC, _C), jnp.int32),
            pltpu.VMEM((_C,), jnp.float32),
            pltpu.VMEM((TR,), jnp.float32),
            pltpu.VMEM_SHARED((NP,), jnp.float32),
            pltpu.SemaphoreType.DMA,
        ],
        compiler_params=pltpu.CompilerParams(use_tc_tiling_on_sc=False),
    )
    def deg_kernel(dst_hbm, ones_hbm, zrow_hbm, out_hbm, dst_v, ones_v, buf_v,
                   acc_sh, ssem):
        c = lax.axis_index("c")
        s = lax.axis_index("s")
        wid = s * _NC + c
        pltpu.sync_copy(zrow_hbm, buf_v)
        pltpu.sync_copy(buf_v, acc_sh.at[pl.ds(s * TR, TR)])
        pltpu.sync_copy(ones_hbm, ones_v)
        pltpu.sync_copy(dst_hbm.at[wid], dst_v)
        plsc.subcore_barrier()

        def fire(j, carry):
            pltpu.async_copy(ones_v, acc_sh.at[dst_v.at[j]], ssem, add=True)
            return carry

        lax.fori_loop(0, KC, fire, 0)

        def drain(j, carry):
            pltpu.make_async_copy(ones_v, acc_sh.at[dst_v.at[0]], ssem).wait()
            return carry

        lax.fori_loop(0, KC, drain, 0)
        plsc.subcore_barrier()
        pltpu.sync_copy(acc_sh.at[pl.ds(s * TR, TR)], buf_v)
        pltpu.sync_copy(buf_v, out_hbm.at[pl.ds(c * NP + s * TR, TR)])

    return deg_kernel


def _make_prop_kernel(NP, KC, H):
    TR = NP // _NS
    G = KC // _NBUF
    RING = _NBUF
    mesh = plsc.VectorSubcoreMesh(core_axis_name="c", subcore_axis_name="s")

    @functools.partial(
        pl.kernel,
        out_type=jax.ShapeDtypeStruct((_NC, NP, H), jnp.float32),
        mesh=mesh,
        scratch_types=[
            pltpu.VMEM((KC, _C), jnp.int32),
            pltpu.VMEM((KC, _C), jnp.int32),
            pltpu.VMEM((RING, _C, H), jnp.float32),
            pltpu.VMEM((TR, H), jnp.float32),
            pltpu.VMEM_SHARED((NP, H), jnp.float32),
            pltpu.SemaphoreType.DMA((RING,)),
            pltpu.SemaphoreType.DMA((RING,)),
        ],
        compiler_params=pltpu.CompilerParams(use_tc_tiling_on_sc=False),
    )
    def prop_kernel(tbl_hbm, src_hbm, dst_hbm, ztile_hbm, out_hbm,
                    src_v, dst_v, rows_v, buf_v, acc_sh, gsem, ssem):
        c = lax.axis_index("c")
        s = lax.axis_index("s")
        wid = s * _NC + c
        pltpu.sync_copy(ztile_hbm, buf_v)
        pltpu.sync_copy(buf_v, acc_sh.at[pl.ds(s * TR, TR)])
        pltpu.sync_copy(src_hbm.at[wid], src_v)
        pltpu.sync_copy(dst_hbm.at[wid], dst_v)
        plsc.subcore_barrier()

        for b in range(_NBUF):
            pltpu.async_copy(tbl_hbm.at[src_v.at[b]], rows_v.at[b], gsem.at[b])

        def body(g, carry):
            for b in range(_NBUF):
                j = g * _NBUF + b
                pltpu.make_async_copy(
                    tbl_hbm.at[src_v.at[j]], rows_v.at[b], gsem.at[b]).wait()
                pltpu.async_copy(
                    rows_v.at[b], acc_sh.at[dst_v.at[j]], ssem.at[b], add=True)

                @pl.when(g < G - 1)
                def _():
                    pltpu.make_async_copy(
                        rows_v.at[b], acc_sh.at[dst_v.at[j]], ssem.at[b]).wait()
                    pltpu.async_copy(
                        tbl_hbm.at[src_v.at[j + _NBUF]], rows_v.at[b], gsem.at[b])
            return carry

        lax.fori_loop(0, G, body, 0)
        for b in range(_NBUF):
            j = KC - _NBUF + b
            pltpu.make_async_copy(
                rows_v.at[b], acc_sh.at[dst_v.at[j]], ssem.at[b]).wait()
        plsc.subcore_barrier()
        pltpu.sync_copy(acc_sh.at[pl.ds(s * TR, TR)], buf_v)
        pltpu.sync_copy(buf_v, out_hbm.at[c, pl.ds(s * TR, TR)])

    return prop_kernel


def _mm1_body(x_ref, w1_ref, dega_ref, degb_ref, dinv_ref, p1_ref):
    d = w1_ref.shape[0]
    dinv = lax.rsqrt(dega_ref[...] + degb_ref[...] + 1.0)
    dinv_ref[...] = dinv
    h = x_ref[:, :d] * x_ref[:, d:d + 1]
    p1_ref[...] = jnp.dot(h, w1_ref[...], preferred_element_type=jnp.float32,
                          precision=lax.Precision.HIGHEST) * dinv


def _layer_body(parts_ref, p1_ref, dinv_ref, b1_ref, w2_ref, q_ref):
    agg = (parts_ref[0] + parts_ref[1] + p1_ref[...]) * dinv_ref[...]
    h1 = jnp.maximum(agg + b1_ref[...], 0.0)
    q_ref[...] = jnp.dot(h1, w2_ref[...], preferred_element_type=jnp.float32,
                         precision=lax.Precision.HIGHEST) * dinv_ref[...]


def _final_body(parts_ref, q_ref, dinv_ref, b2_ref, wd_ref, bd_ref, out_ref):
    i = pl.program_id(0)
    agg = (parts_ref[0] + parts_ref[1] + q_ref[...]) * dinv_ref[...]
    h2 = jnp.maximum(agg + b2_ref[...], 0.0)
    pooled = jnp.sum(h2, axis=0, keepdims=True)
    val = jnp.dot(pooled, wd_ref[...], preferred_element_type=jnp.float32,
                  precision=lax.Precision.HIGHEST)

    @pl.when(i == 0)
    def _():
        out_ref[...] = bd_ref[...]

    out_ref[...] += val


def kernel(x, edge_index, W1, b1, W2, b2, Wd, bd):
    n = x.shape[0]
    d = W1.shape[0]
    H = W1.shape[1]
    e = edge_index.shape[1]

    NP = -(-n // 128) * 128
    EPT = -(-e // _NW)
    KC = -(-(-(-EPT // _C)) // _NBUF) * _NBUF
    EPTP = KC * _C
    NG = n // _RB

    ei = edge_index.astype(jnp.int32)
    if _NW * EPT != e:
        pad = jnp.concatenate([jnp.zeros((1, _NW * EPT - e), jnp.int32),
                               jnp.full((1, _NW * EPT - e), n, jnp.int32)])
        ei = jnp.concatenate([ei, pad], axis=1)
    ei3 = ei.reshape(2, _NW, EPT)
    if EPTP != EPT:
        ntrash = NP - n
        j = jnp.arange(EPTP - EPT, dtype=jnp.int32)[None, :]
        w = jnp.arange(_NW, dtype=jnp.int32)[:, None]
        dpad = n + (j + w * 7) % ntrash
        padt = jnp.stack([jnp.zeros((_NW, EPTP - EPT), jnp.int32), dpad])
        ei3 = jnp.concatenate([ei3, padt], axis=2)
    src3 = ei3[0].reshape(_NW, KC, _C)
    dst3 = ei3[1].reshape(_NW, KC, _C)

    ones_c = jnp.ones((_C,), jnp.float32)
    zrow = jnp.zeros((NP // _NS,), jnp.float32)
    ztile = jnp.zeros((NP // _NS, H), jnp.float32)

    deg_kernel = _make_deg_kernel(NP, KC)
    prop_kernel = _make_prop_kernel(NP, KC, H)

    deg1d = deg_kernel(dst3, ones_c, zrow)
    dega = deg1d[:NP].reshape(NP, 1)
    degb = deg1d[NP:].reshape(NP, 1)

    dinv, p1 = pl.pallas_call(
        _mm1_body,
        grid=(NG,),
        in_specs=[pl.BlockSpec((_RB, x.shape[1]), lambda i: (i, 0)),
                  pl.BlockSpec((d, H), lambda i: (0, 0)),
                  pl.BlockSpec((_RB, 1), lambda i: (i, 0)),
                  pl.BlockSpec((_RB, 1), lambda i: (i, 0))],
        out_specs=(pl.BlockSpec((_RB, 1), lambda i: (i, 0)),
                   pl.BlockSpec((_RB, H), lambda i: (i, 0))),
        out_shape=(jax.ShapeDtypeStruct((NP, 1), jnp.float32),
                   jax.ShapeDtypeStruct((NP, H), jnp.float32)),
    )(x, W1, dega, degb)

    parts1 = prop_kernel(p1, src3, dst3, ztile)

    q = pl.pallas_call(
        _layer_body,
        grid=(NG,),
        in_specs=[pl.BlockSpec((2, _RB, H), lambda i: (0, i, 0)),
                  pl.BlockSpec((_RB, H), lambda i: (i, 0)),
                  pl.BlockSpec((_RB, 1), lambda i: (i, 0)),
                  pl.BlockSpec((1, H), lambda i: (0, 0)),
                  pl.BlockSpec((H, H), lambda i: (0, 0))],
        out_specs=pl.BlockSpec((_RB, H), lambda i: (i, 0)),
        out_shape=jax.ShapeDtypeStruct((NP, H), jnp.float32),
    )(parts1, p1, dinv, b1.reshape(1, H), W2)

    parts2 = prop_kernel(q, src3, dst3, ztile)

    out = pl.pallas_call(
        _final_body,
        grid=(NG,),
        in_specs=[pl.BlockSpec((2, _RB, H), lambda i: (0, i, 0)),
                  pl.BlockSpec((_RB, H), lambda i: (i, 0)),
                  pl.BlockSpec((_RB, 1), lambda i: (i, 0)),
                  pl.BlockSpec((1, H), lambda i: (0, 0)),
                  pl.BlockSpec((H, 1), lambda i: (0, 0)),
                  pl.BlockSpec((1, 1), lambda i: (0, 0))],
        out_specs=pl.BlockSpec((1, 1), lambda i: (0, 0)),
        out_shape=jax.ShapeDtypeStruct((1, 1), jnp.float32),
    )(parts2, q, dinv, b2.reshape(1, H), Wd, bd.reshape(1, 1))

    return out


# speedup vs baseline: 1.6179x; 1.6050x over previous
import functools

import jax
import jax.numpy as jnp
from jax import lax
from jax.experimental import pallas as pl
from jax.experimental.pallas import tpu as pltpu
from jax.experimental.pallas import tpu_sc as plsc

_NC = 2
_NS = 16
_NW = _NC * _NS
_C = 80
_NBUF = 5
_RB = 1000


def _make_deg_kernel(NP, KC):
    TR = NP // _NS
    mesh = plsc.VectorSubcoreMesh(core_axis_name="c", subcore_axis_name="s")

    @functools.partial(
        pl.kernel,
        out_type=jax.ShapeDtypeStruct((_NC * NP,), jnp.float32),
        mesh=mesh,
        scratch_types=[
            pltpu.VMEM((KC, _C), jnp.int32),
            pltpu.VMEM((_C,), jnp.float32),
            pltpu.VMEM((TR,), jnp.float32),
            pltpu.VMEM_SHARED((NP,), jnp.float32),
            pltpu.SemaphoreType.DMA,
        ],
        compiler_params=pltpu.CompilerParams(use_tc_tiling_on_sc=False),
    )
    def deg_kernel(dst_hbm, ones_hbm, zrow_hbm, out_hbm, dst_v, ones_v, buf_v,
                   acc_sh, ssem):
        c = lax.axis_index("c")
        s = lax.axis_index("s")
        wid = s * _NC + c
        pltpu.sync_copy(zrow_hbm, buf_v)
        pltpu.sync_copy(buf_v, acc_sh.at[pl.ds(s * TR, TR)])
        pltpu.sync_copy(ones_hbm, ones_v)
        pltpu.sync_copy(dst_hbm.at[wid], dst_v)
        plsc.subcore_barrier()

        def fire(j, carry):
            pltpu.async_copy(ones_v, acc_sh.at[dst_v.at[j]], ssem, add=True)
            return carry

        lax.fori_loop(0, KC, fire, 0)

        def drain(j, carry):
            pltpu.make_async_copy(ones_v, acc_sh.at[dst_v.at[0]], ssem).wait()
            return carry

        lax.fori_loop(0, KC, drain, 0)
        plsc.subcore_barrier()
        pltpu.sync_copy(acc_sh.at[pl.ds(s * TR, TR)], buf_v)
        pltpu.sync_copy(buf_v, out_hbm.at[pl.ds(c * NP + s * TR, TR)])

    return deg_kernel


def _make_prop_kernel(NP, KC, H):
    TR = NP // _NS
    G = KC // _NBUF
    RING = _NBUF
    mesh = plsc.VectorSubcoreMesh(core_axis_name="c", subcore_axis_name="s")

    @functools.partial(
        pl.kernel,
        out_type=jax.ShapeDtypeStruct((_NC, NP, H), jnp.float32),
        mesh=mesh,
        scratch_types=[
            pltpu.VMEM((KC, _C), jnp.int32),
            pltpu.VMEM((KC, _C), jnp.int32),
            pltpu.VMEM((RING, _C, H), jnp.float32),
            pltpu.VMEM((TR, H), jnp.float32),
            pltpu.VMEM_SHARED((NP, H), jnp.float32),
            pltpu.SemaphoreType.DMA((RING,)),
            pltpu.SemaphoreType.DMA((RING,)),
        ],
        compiler_params=pltpu.CompilerParams(use_tc_tiling_on_sc=False),
    )
    def prop_kernel(tbl_hbm, src_hbm, dst_hbm, ztile_hbm, out_hbm,
                    src_v, dst_v, rows_v, buf_v, acc_sh, gsem, ssem):
        c = lax.axis_index("c")
        s = lax.axis_index("s")
        wid = s * _NC + c
        pltpu.sync_copy(ztile_hbm, buf_v)
        pltpu.sync_copy(buf_v, acc_sh.at[pl.ds(s * TR, TR)])
        pltpu.sync_copy(src_hbm.at[wid], src_v)
        pltpu.sync_copy(dst_hbm.at[wid], dst_v)
        plsc.subcore_barrier()

        for b in range(_NBUF):
            pltpu.async_copy(tbl_hbm.at[src_v.at[b]], rows_v.at[b], gsem.at[b])

        def body(g, carry):
            for b in range(_NBUF):
                j = g * _NBUF + b
                pltpu.make_async_copy(
                    tbl_hbm.at[src_v.at[j]], rows_v.at[b], gsem.at[b]).wait()
                pltpu.async_copy(
                    rows_v.at[b], acc_sh.at[dst_v.at[j]], ssem.at[b], add=True)

                @pl.when(g < G - 1)
                def _():
                    pltpu.make_async_copy(
                        rows_v.at[b], acc_sh.at[dst_v.at[j]], ssem.at[b]).wait()
                    pltpu.async_copy(
                        tbl_hbm.at[src_v.at[j + _NBUF]], rows_v.at[b], gsem.at[b])
            return carry

        lax.fori_loop(0, G, body, 0)
        for b in range(_NBUF):
            j = KC - _NBUF + b
            pltpu.make_async_copy(
                rows_v.at[b], acc_sh.at[dst_v.at[j]], ssem.at[b]).wait()
        plsc.subcore_barrier()
        pltpu.sync_copy(acc_sh.at[pl.ds(s * TR, TR)], buf_v)
        pltpu.sync_copy(buf_v, out_hbm.at[c, pl.ds(s * TR, TR)])

    return prop_kernel


def _mm1_body(x_ref, w1_ref, dega_ref, degb_ref, dinv_ref, p1_ref):
    d = w1_ref.shape[0]
    dinv = lax.rsqrt(dega_ref[...] + degb_ref[...] + 1.0)
    dinv_ref[...] = dinv
    h = x_ref[:, :d] * x_ref[:, d:d + 1]
    p1_ref[...] = jnp.dot(h, w1_ref[...], preferred_element_type=jnp.float32,
                          precision=lax.Precision.HIGHEST) * dinv


def _layer_body(parts_ref, p1_ref, dinv_ref, b1_ref, w2_ref, q_ref):
    agg = (parts_ref[0] + parts_ref[1] + p1_ref[...]) * dinv_ref[...]
    h1 = jnp.maximum(agg + b1_ref[...], 0.0)
    q_ref[...] = jnp.dot(h1, w2_ref[...], preferred_element_type=jnp.float32,
                         precision=lax.Precision.HIGHEST) * dinv_ref[...]


def _final_body(parts_ref, q_ref, dinv_ref, b2_ref, wd_ref, bd_ref, out_ref):
    i = pl.program_id(0)
    agg = (parts_ref[0] + parts_ref[1] + q_ref[...]) * dinv_ref[...]
    h2 = jnp.maximum(agg + b2_ref[...], 0.0)
    pooled = jnp.sum(h2, axis=0, keepdims=True)
    val = jnp.dot(pooled, wd_ref[...], preferred_element_type=jnp.float32,
                  precision=lax.Precision.HIGHEST)

    @pl.when(i == 0)
    def _():
        out_ref[...] = bd_ref[...]

    out_ref[...] += val


def kernel(x, edge_index, W1, b1, W2, b2, Wd, bd):
    n = x.shape[0]
    d = W1.shape[0]
    H = W1.shape[1]
    e = edge_index.shape[1]

    NP = -(-n // 128) * 128
    EPT = -(-e // _NW)
    KC = -(-(-(-EPT // _C)) // _NBUF) * _NBUF
    EPTP = KC * _C
    NG = n // _RB

    ei = edge_index.astype(jnp.int32)
    if _NW * EPT != e:
        pad = jnp.concatenate([jnp.zeros((1, _NW * EPT - e), jnp.int32),
                               jnp.full((1, _NW * EPT - e), n, jnp.int32)])
        ei = jnp.concatenate([ei, pad], axis=1)
    ei3 = ei.reshape(2, _NW, EPT)
    if EPTP != EPT:
        ntrash = NP - n
        j = jnp.arange(EPTP - EPT, dtype=jnp.int32)[None, :]
        w = jnp.arange(_NW, dtype=jnp.int32)[:, None]
        dpad = n + (j + w * 7) % ntrash
        padt = jnp.stack([jnp.zeros((_NW, EPTP - EPT), jnp.int32), dpad])
        ei3 = jnp.concatenate([ei3, padt], axis=2)
    src3 = ei3[0].reshape(_NW, KC, _C)
    dst3 = ei3[1].reshape(_NW, KC, _C)

    ones_c = jnp.ones((_C,), jnp.float32)
    zrow = jnp.zeros((NP // _NS,), jnp.float32)
    ztile = jnp.zeros((NP // _NS, H), jnp.float32)

    deg_kernel = _make_deg_kernel(NP, KC)
    prop_kernel = _make_prop_kernel(NP, KC, H)

    deg1d = deg_kernel(dst3, ones_c, zrow)
    dega = deg1d[:NP].reshape(NP, 1)
    degb = deg1d[NP:].reshape(NP, 1)

    dinv, p1 = pl.pallas_call(
        _mm1_body,
        grid=(NG,),
        in_specs=[pl.BlockSpec((_RB, x.shape[1]), lambda i: (i, 0)),
                  pl.BlockSpec((d, H), lambda i: (0, 0)),
                  pl.BlockSpec((_RB, 1), lambda i: (i, 0)),
                  pl.BlockSpec((_RB, 1), lambda i: (i, 0))],
        out_specs=(pl.BlockSpec((_RB, 1), lambda i: (i, 0)),
                   pl.BlockSpec((_RB, H), lambda i: (i, 0))),
        out_shape=(jax.ShapeDtypeStruct((NP, 1), jnp.float32),
                   jax.ShapeDtypeStruct((NP, H), jnp.float32)),
    )(x, W1, dega, degb)

    parts1 = prop_kernel(p1, src3, dst3, ztile)

    q = pl.pallas_call(
        _layer_body,
        grid=(NG,),
        in_specs=[pl.BlockSpec((2, _RB, H), lambda i: (0, i, 0)),
                  pl.BlockSpec((_RB, H), lambda i: (i, 0)),
                  pl.BlockSpec((_RB, 1), lambda i: (i, 0)),
                  pl.BlockSpec((1, H), lambda i: (0, 0)),
                  pl.BlockSpec((H, H), lambda i: (0, 0))],
        out_specs=pl.BlockSpec((_RB, H), lambda i: (i, 0)),
        out_shape=jax.ShapeDtypeStruct((NP, H), jnp.float32),
    )(parts1, p1, dinv, b1.reshape(1, H), W2)

    parts2 = prop_kernel(q, src3, dst3, ztile)

    out = pl.pallas_call(
        _final_body,
        grid=(NG,),
        in_specs=[pl.BlockSpec((2, _RB, H), lambda i: (0, i, 0)),
                  pl.BlockSpec((_RB, H), lambda i: (i, 0)),
                  pl.BlockSpec((_RB, 1), lambda i: (i, 0)),
                  pl.BlockSpec((1, H), lambda i: (0, 0)),
                  pl.BlockSpec((H, 1), lambda i: (0, 0)),
                  pl.BlockSpec((1, 1), lambda i: (0, 0))],
        out_specs=pl.BlockSpec((1, 1), lambda i: (0, 0)),
        out_shape=jax.ShapeDtypeStruct((1, 1), jnp.float32),
    )(parts2, q, dinv, b2.reshape(1, H), Wd, bd.reshape(1, 1))

    return out


# speedup vs baseline: 1.6966x; 1.0487x over previous
import functools

import jax
import jax.numpy as jnp
from jax import lax
from jax.experimental import pallas as pl
from jax.experimental.pallas import tpu as pltpu
from jax.experimental.pallas import tpu_sc as plsc

_NC = 2
_NS = 16
_NW = _NC * _NS
_C = 80
_NBUF = 5


def _make_deg_kernel(NP, KC):
    TR = NP // _NS
    mesh = plsc.VectorSubcoreMesh(core_axis_name="c", subcore_axis_name="s")

    @functools.partial(
        pl.kernel,
        out_type=jax.ShapeDtypeStruct((_NC * NP,), jnp.float32),
        mesh=mesh,
        scratch_types=[
            pltpu.VMEM((KC, _C), jnp.int32),
            pltpu.VMEM((_C,), jnp.float32),
            pltpu.VMEM((TR,), jnp.float32),
            pltpu.VMEM_SHARED((NP,), jnp.float32),
            pltpu.SemaphoreType.DMA,
        ],
        compiler_params=pltpu.CompilerParams(use_tc_tiling_on_sc=False),
    )
    def deg_kernel(dst_hbm, ones_hbm, zrow_hbm, out_hbm, dst_v, ones_v, buf_v,
                   acc_sh, ssem):
        c = lax.axis_index("c")
        s = lax.axis_index("s")
        wid = s * _NC + c
        pltpu.sync_copy(zrow_hbm, buf_v)
        pltpu.sync_copy(buf_v, acc_sh.at[pl.ds(s * TR, TR)])
        pltpu.sync_copy(ones_hbm, ones_v)
        pltpu.sync_copy(dst_hbm.at[wid], dst_v)
        plsc.subcore_barrier()

        def fire(j, carry):
            pltpu.async_copy(ones_v, acc_sh.at[dst_v.at[j]], ssem, add=True)
            return carry

        lax.fori_loop(0, KC, fire, 0)

        def drain(j, carry):
            pltpu.make_async_copy(ones_v, acc_sh.at[dst_v.at[0]], ssem).wait()
            return carry

        lax.fori_loop(0, KC, drain, 0)
        plsc.subcore_barrier()
        pltpu.sync_copy(acc_sh.at[pl.ds(s * TR, TR)], buf_v)
        pltpu.sync_copy(buf_v, out_hbm.at[pl.ds(c * NP + s * TR, TR)])

    return deg_kernel


def _make_prop_kernel(NP, KC, H):
    TR = NP // _NS
    G = KC // _NBUF
    RING = _NBUF
    mesh = plsc.VectorSubcoreMesh(core_axis_name="c", subcore_axis_name="s")

    @functools.partial(
        pl.kernel,
        out_type=jax.ShapeDtypeStruct((_NC, NP, H), jnp.float32),
        mesh=mesh,
        scratch_types=[
            pltpu.VMEM((KC, _C), jnp.int32),
            pltpu.VMEM((KC, _C), jnp.int32),
            pltpu.VMEM((RING, _C, H), jnp.float32),
            pltpu.VMEM((TR, H), jnp.float32),
            pltpu.VMEM_SHARED((NP, H), jnp.float32),
            pltpu.SemaphoreType.DMA((RING,)),
            pltpu.SemaphoreType.DMA((RING,)),
        ],
        compiler_params=pltpu.CompilerParams(use_tc_tiling_on_sc=False),
    )
    def prop_kernel(tbl_hbm, src_hbm, dst_hbm, ztile_hbm, out_hbm,
                    src_v, dst_v, rows_v, buf_v, acc_sh, gsem, ssem):
        c = lax.axis_index("c")
        s = lax.axis_index("s")
        wid = s * _NC + c
        pltpu.sync_copy(ztile_hbm, buf_v)
        pltpu.sync_copy(buf_v, acc_sh.at[pl.ds(s * TR, TR)])
        pltpu.sync_copy(src_hbm.at[wid], src_v)
        pltpu.sync_copy(dst_hbm.at[wid], dst_v)
        plsc.subcore_barrier()

        for b in range(_NBUF):
            pltpu.async_copy(tbl_hbm.at[src_v.at[b]], rows_v.at[b], gsem.at[b])

        def body(g, carry):
            for b in range(_NBUF):
                j = g * _NBUF + b
                pltpu.make_async_copy(
                    tbl_hbm.at[src_v.at[j]], rows_v.at[b], gsem.at[b]).wait()
                pltpu.async_copy(
                    rows_v.at[b], acc_sh.at[dst_v.at[j]], ssem.at[b], add=True)

                @pl.when(g < G - 1)
                def _():
                    pltpu.make_async_copy(
                        rows_v.at[b], acc_sh.at[dst_v.at[j]], ssem.at[b]).wait()
                    pltpu.async_copy(
                        tbl_hbm.at[src_v.at[j + _NBUF]], rows_v.at[b], gsem.at[b])
            return carry

        lax.fori_loop(0, G, body, 0)
        for b in range(_NBUF):
            j = KC - _NBUF + b
            pltpu.make_async_copy(
                rows_v.at[b], acc_sh.at[dst_v.at[j]], ssem.at[b]).wait()
        plsc.subcore_barrier()
        pltpu.sync_copy(acc_sh.at[pl.ds(s * TR, TR)], buf_v)
        pltpu.sync_copy(buf_v, out_hbm.at[c, pl.ds(s * TR, TR)])

    return prop_kernel


def _mm1_body(feat_ref, mask_ref, w1_ref, dega_ref, degb_ref, dinv_ref, p1_ref):
    dinv = lax.rsqrt(dega_ref[...] + degb_ref[...] + 1.0)
    dinv_ref[...] = dinv
    p1_ref[...] = jnp.dot(feat_ref[...], w1_ref[...],
                          preferred_element_type=jnp.float32,
                          precision=lax.Precision.HIGHEST) * (mask_ref[...] * dinv)


def _layer_body(parts_ref, p1_ref, dinv_ref, b1_ref, w2_ref, q_ref):
    n = p1_ref.shape[0]
    agg = (parts_ref[0, :n] + parts_ref[1, :n] + p1_ref[...]) * dinv_ref[...]
    h1 = jnp.maximum(agg + b1_ref[...], 0.0)
    q_ref[...] = jnp.dot(h1, w2_ref[...], preferred_element_type=jnp.float32,
                         precision=lax.Precision.HIGHEST) * dinv_ref[...]


def _final_body(parts_ref, q_ref, dinv_ref, b2_ref, wd_ref, bd_ref, out_ref):
    n = q_ref.shape[0]
    agg = (parts_ref[0, :n] + parts_ref[1, :n] + q_ref[...]) * dinv_ref[...]
    h2 = jnp.maximum(agg + b2_ref[...], 0.0)
    pooled = jnp.sum(h2, axis=0, keepdims=True)
    out_ref[...] = jnp.dot(pooled, wd_ref[...], preferred_element_type=jnp.float32,
                           precision=lax.Precision.HIGHEST) + bd_ref[...]


def kernel(x, edge_index, W1, b1, W2, b2, Wd, bd):
    n = x.shape[0]
    d = W1.shape[0]
    H = W1.shape[1]
    e = edge_index.shape[1]

    NP = -(-n // 128) * 128
    EPT = -(-e // _NW)
    KC = -(-(-(-EPT // _C)) // _NBUF) * _NBUF
    EPTP = KC * _C

    ei = edge_index.astype(jnp.int32)
    if _NW * EPT != e:
        pad = jnp.concatenate([jnp.zeros((1, _NW * EPT - e), jnp.int32),
                               jnp.full((1, _NW * EPT - e), n, jnp.int32)])
        ei = jnp.concatenate([ei, pad], axis=1)
    ei3 = ei.reshape(2, _NW, EPT)
    if EPTP != EPT:
        ntrash = NP - n
        j = jnp.arange(EPTP - EPT, dtype=jnp.int32)[None, :]
        w = jnp.arange(_NW, dtype=jnp.int32)[:, None]
        dpad = n + (j + w * 7) % ntrash
        padt = jnp.stack([jnp.zeros((_NW, EPTP - EPT), jnp.int32), dpad])
        ei3 = jnp.concatenate([ei3, padt], axis=2)
    src3 = ei3[0].reshape(_NW, KC, _C)
    dst3 = ei3[1].reshape(_NW, KC, _C)

    ones_c = jnp.ones((_C,), jnp.float32)
    zrow = jnp.zeros((NP // _NS,), jnp.float32)
    ztile = jnp.zeros((NP // _NS, H), jnp.float32)

    deg_kernel = _make_deg_kernel(NP, KC)
    prop_kernel = _make_prop_kernel(NP, KC, H)

    deg1d = deg_kernel(dst3, ones_c, zrow)
    dega = deg1d[:n].reshape(n, 1)
    degb = deg1d[NP:NP + n].reshape(n, 1)
    feat = x[:, :d]
    mask = x[:, d:d + 1]

    dinv, p1 = pl.pallas_call(
        _mm1_body,
        out_shape=(jax.ShapeDtypeStruct((n, 1), jnp.float32),
                   jax.ShapeDtypeStruct((n, H), jnp.float32)),
    )(feat, mask, W1, dega, degb)

    parts1 = prop_kernel(p1, src3, dst3, ztile)

    q = pl.pallas_call(
        _layer_body,
        out_shape=jax.ShapeDtypeStruct((n, H), jnp.float32),
    )(parts1, p1, dinv, b1.reshape(1, H), W2)

    parts2 = prop_kernel(q, src3, dst3, ztile)

    out = pl.pallas_call(
        _final_body,
        out_shape=jax.ShapeDtypeStruct((1, 1), jnp.float32),
    )(parts2, q, dinv, b2.reshape(1, H), Wd, bd.reshape(1, 1))

    return out


# speedup vs baseline: 1.8776x; 1.1066x over previous
import functools

import jax
import jax.numpy as jnp
from jax import lax
from jax.experimental import pallas as pl
from jax.experimental.pallas import tpu as pltpu
from jax.experimental.pallas import tpu_sc as plsc

_NC = 2
_NS = 16
_NW = _NC * _NS
_C = 80
_NBUF = 5


def _make_deg_kernel(NP, KC):
    TR = NP // _NS
    mesh = plsc.VectorSubcoreMesh(core_axis_name="c", subcore_axis_name="s")

    @functools.partial(
        pl.kernel,
        out_type=jax.ShapeDtypeStruct((_NC * NP,), jnp.float32),
        mesh=mesh,
        scratch_types=[
            pltpu.VMEM((KC, _C), jnp.int32),
            pltpu.VMEM((_C,), jnp.float32),
            pltpu.VMEM((TR,), jnp.float32),
            pltpu.VMEM_SHARED((NP,), jnp.float32),
            pltpu.SemaphoreType.DMA,
        ],
        compiler_params=pltpu.CompilerParams(use_tc_tiling_on_sc=False),
    )
    def deg_kernel(dst_hbm, ones_hbm, zrow_hbm, out_hbm, dst_v, ones_v, buf_v,
                   acc_sh, ssem):
        c = lax.axis_index("c")
        s = lax.axis_index("s")
        wid = s * _NC + c
        pltpu.sync_copy(zrow_hbm, buf_v)
        pltpu.sync_copy(buf_v, acc_sh.at[pl.ds(s * TR, TR)])
        pltpu.sync_copy(ones_hbm, ones_v)
        pltpu.sync_copy(dst_hbm.at[wid], dst_v)
        plsc.subcore_barrier()

        def fire(j, carry):
            pltpu.async_copy(ones_v, acc_sh.at[dst_v.at[j]], ssem, add=True)
            return carry

        lax.fori_loop(0, KC, fire, 0)

        def drain(j, carry):
            pltpu.make_async_copy(ones_v, acc_sh.at[dst_v.at[0]], ssem).wait()
            return carry

        lax.fori_loop(0, KC, drain, 0)
        plsc.subcore_barrier()
        pltpu.sync_copy(acc_sh.at[pl.ds(s * TR, TR)], buf_v)
        pltpu.sync_copy(buf_v, out_hbm.at[pl.ds(c * NP + s * TR, TR)])

    return deg_kernel


def _make_prop_kernel(NP, KC, H):
    TR = NP // _NS
    G = KC // _NBUF
    RING = _NBUF
    mesh = plsc.VectorSubcoreMesh(core_axis_name="c", subcore_axis_name="s")

    @functools.partial(
        pl.kernel,
        out_type=jax.ShapeDtypeStruct((_NC, NP, H), jnp.float32),
        mesh=mesh,
        scratch_types=[
            pltpu.VMEM((KC, _C), jnp.int32),
            pltpu.VMEM((KC, _C), jnp.int32),
            pltpu.VMEM((RING, _C, H), jnp.float32),
            pltpu.VMEM((TR, H), jnp.float32),
            pltpu.VMEM_SHARED((NP, H), jnp.float32),
            pltpu.SemaphoreType.DMA((RING,)),
            pltpu.SemaphoreType.DMA((RING,)),
        ],
        compiler_params=pltpu.CompilerParams(use_tc_tiling_on_sc=False),
    )
    def prop_kernel(tbl_hbm, src_hbm, dst_hbm, ztile_hbm, out_hbm,
                    src_v, dst_v, rows_v, buf_v, acc_sh, gsem, ssem):
        c = lax.axis_index("c")
        s = lax.axis_index("s")
        wid = s * _NC + c
        pltpu.sync_copy(ztile_hbm, buf_v)
        pltpu.sync_copy(buf_v, acc_sh.at[pl.ds(s * TR, TR)])
        pltpu.sync_copy(src_hbm.at[wid], src_v)
        pltpu.sync_copy(dst_hbm.at[wid], dst_v)
        plsc.subcore_barrier()

        for b in range(_NBUF):
            pltpu.async_copy(tbl_hbm.at[src_v.at[b]], rows_v.at[b], gsem.at[b])

        def body(g, carry):
            for b in range(_NBUF):
                j = g * _NBUF + b
                pltpu.make_async_copy(
                    tbl_hbm.at[src_v.at[j]], rows_v.at[b], gsem.at[b]).wait()
                pltpu.async_copy(
                    rows_v.at[b], acc_sh.at[dst_v.at[j]], ssem.at[b], add=True)

                @pl.when(g < G - 1)
                def _():
                    pltpu.make_async_copy(
                        rows_v.at[b], acc_sh.at[dst_v.at[j]], ssem.at[b]).wait()
                    pltpu.async_copy(
                        tbl_hbm.at[src_v.at[j + _NBUF]], rows_v.at[b], gsem.at[b])
            return carry

        lax.fori_loop(0, G, body, 0)
        for b in range(_NBUF):
            j = KC - _NBUF + b
            pltpu.make_async_copy(
                rows_v.at[b], acc_sh.at[dst_v.at[j]], ssem.at[b]).wait()
        plsc.subcore_barrier()
        pltpu.sync_copy(acc_sh.at[pl.ds(s * TR, TR)], buf_v)
        pltpu.sync_copy(buf_v, out_hbm.at[c, pl.ds(s * TR, TR)])

    return prop_kernel


def _mm1_body(NP, feat_ref, mask_ref, w1_ref, deg_ref, dinv_ref, p1_ref):
    n = feat_ref.shape[0]
    deg = deg_ref[:n] + deg_ref[NP:NP + n]
    dinv = lax.rsqrt(deg + 1.0)
    dinv_ref[...] = dinv
    m = jnp.reshape(mask_ref[...] * dinv, (n, 1))
    p1_ref[...] = jnp.dot(feat_ref[...], w1_ref[...],
                          preferred_element_type=jnp.float32,
                          precision=lax.Precision.HIGHEST) * m


def _layer_body(parts_ref, p1_ref, dinv_ref, b1_ref, w2_ref, q_ref):
    n = p1_ref.shape[0]
    dcol = jnp.reshape(dinv_ref[...], (n, 1))
    agg = (parts_ref[0, :n] + parts_ref[1, :n] + p1_ref[...]) * dcol
    h1 = jnp.maximum(agg + b1_ref[...], 0.0)
    q_ref[...] = jnp.dot(h1, w2_ref[...], preferred_element_type=jnp.float32,
                         precision=lax.Precision.HIGHEST) * dcol


def _final_body(parts_ref, q_ref, dinv_ref, b2_ref, wd_ref, bd_ref, out_ref):
    n = q_ref.shape[0]
    dcol = jnp.reshape(dinv_ref[...], (n, 1))
    agg = (parts_ref[0, :n] + parts_ref[1, :n] + q_ref[...]) * dcol
    h2 = jnp.maximum(agg + b2_ref[...], 0.0)
    pooled = jnp.sum(h2, axis=0, keepdims=True)
    out_ref[...] = jnp.dot(pooled, wd_ref[...], preferred_element_type=jnp.float32,
                           precision=lax.Precision.HIGHEST) + bd_ref[...]


def kernel(x, edge_index, W1, b1, W2, b2, Wd, bd):
    n = x.shape[0]
    d = W1.shape[0]
    H = W1.shape[1]
    e = edge_index.shape[1]

    NP = -(-n // 128) * 128
    EPT = -(-e // _NW)
    KC = -(-(-(-EPT // _C)) // _NBUF) * _NBUF
    EPTP = KC * _C

    ei = edge_index.astype(jnp.int32)
    if _NW * EPT != e:
        pad = jnp.concatenate([jnp.zeros((1, _NW * EPT - e), jnp.int32),
                               jnp.full((1, _NW * EPT - e), n, jnp.int32)])
        ei = jnp.concatenate([ei, pad], axis=1)
    ei3 = ei.reshape(2, _NW, EPT)
    if EPTP != EPT:
        ntrash = NP - n
        j = jnp.arange(EPTP - EPT, dtype=jnp.int32)[None, :]
        w = jnp.arange(_NW, dtype=jnp.int32)[:, None]
        dpad = n + (j + w * 7) % ntrash
        padt = jnp.stack([jnp.zeros((_NW, EPTP - EPT), jnp.int32), dpad])
        ei3 = jnp.concatenate([ei3, padt], axis=2)
    src3 = ei3[0].reshape(_NW, KC, _C)
    dst3 = ei3[1].reshape(_NW, KC, _C)

    ones_c = jnp.ones((_C,), jnp.float32)
    zrow = jnp.zeros((NP // _NS,), jnp.float32)
    ztile = jnp.zeros((NP // _NS, H), jnp.float32)

    deg_kernel = _make_deg_kernel(NP, KC)
    prop_kernel = _make_prop_kernel(NP, KC, H)

    deg1d = deg_kernel(dst3, ones_c, zrow)
    feat = x[:, :d]
    mask = x[:, d]

    dinv, p1 = pl.pallas_call(
        functools.partial(_mm1_body, NP),
        out_shape=(jax.ShapeDtypeStruct((n,), jnp.float32),
                   jax.ShapeDtypeStruct((n, H), jnp.float32)),
    )(feat, mask, W1, deg1d)

    parts1 = prop_kernel(p1, src3, dst3, ztile)

    q = pl.pallas_call(
        _layer_body,
        out_shape=jax.ShapeDtypeStruct((n, H), jnp.float32),
    )(parts1, p1, dinv, b1, W2)

    parts2 = prop_kernel(q, src3, dst3, ztile)

    out = pl.pallas_call(
        _final_body,
        out_shape=jax.ShapeDtypeStruct((1, 1), jnp.float32),
    )(parts2, q, dinv, b2, Wd, bd.reshape(1, 1))

    return out


# speedup vs baseline: 1.8783x; 1.0004x over previous
import functools

import jax
import jax.numpy as jnp
from jax import lax
from jax.experimental import pallas as pl
from jax.experimental.pallas import tpu as pltpu
from jax.experimental.pallas import tpu_sc as plsc

_NC = 2
_NS = 16
_NW = _NC * _NS
_C = 80
_NBUF = 5


def _make_deg_kernel(NP, KC):
    TR = NP // _NS
    mesh = plsc.VectorSubcoreMesh(core_axis_name="c", subcore_axis_name="s")

    @functools.partial(
        pl.kernel,
        out_type=jax.ShapeDtypeStruct((_NC * NP,), jnp.float32),
        mesh=mesh,
        scratch_types=[
            pltpu.VMEM((KC, _C), jnp.int32),
            pltpu.VMEM((_C,), jnp.float32),
            pltpu.VMEM((TR,), jnp.float32),
            pltpu.VMEM_SHARED((NP,), jnp.float32),
            pltpu.SemaphoreType.DMA,
        ],
        compiler_params=pltpu.CompilerParams(use_tc_tiling_on_sc=False),
    )
    def deg_kernel(dst_hbm, ones_hbm, zrow_hbm, out_hbm, dst_v, ones_v, buf_v,
                   acc_sh, ssem):
        c = lax.axis_index("c")
        s = lax.axis_index("s")
        wid = s * _NC + c
        pltpu.sync_copy(zrow_hbm, buf_v)
        pltpu.sync_copy(buf_v, acc_sh.at[pl.ds(s * TR, TR)])
        pltpu.sync_copy(ones_hbm, ones_v)
        pltpu.sync_copy(dst_hbm.at[wid], dst_v)
        plsc.subcore_barrier()

        def fire(j, carry):
            pltpu.async_copy(ones_v, acc_sh.at[dst_v.at[j]], ssem, add=True)
            pltpu.make_async_copy(ones_v, acc_sh.at[dst_v.at[j]], ssem).wait()
            return carry

        lax.fori_loop(0, KC, fire, 0)
        plsc.subcore_barrier()
        pltpu.sync_copy(acc_sh.at[pl.ds(s * TR, TR)], buf_v)
        pltpu.sync_copy(buf_v, out_hbm.at[pl.ds(c * NP + s * TR, TR)])

    return deg_kernel


def _make_prop_kernel(NP, KC, H):
    TR = NP // _NS
    G = KC // _NBUF
    RING = _NBUF
    mesh = plsc.VectorSubcoreMesh(core_axis_name="c", subcore_axis_name="s")

    @functools.partial(
        pl.kernel,
        out_type=jax.ShapeDtypeStruct((_NC, NP, H), jnp.float32),
        mesh=mesh,
        scratch_types=[
            pltpu.VMEM((KC, _C), jnp.int32),
            pltpu.VMEM((KC, _C), jnp.int32),
            pltpu.VMEM((RING, _C, H), jnp.float32),
            pltpu.VMEM((TR, H), jnp.float32),
            pltpu.VMEM_SHARED((NP, H), jnp.float32),
            pltpu.SemaphoreType.DMA((RING,)),
            pltpu.SemaphoreType.DMA((RING,)),
        ],
        compiler_params=pltpu.CompilerParams(use_tc_tiling_on_sc=False),
    )
    def prop_kernel(tbl_hbm, src_hbm, dst_hbm, ztile_hbm, out_hbm,
                    src_v, dst_v, rows_v, buf_v, acc_sh, gsem, ssem):
        c = lax.axis_index("c")
        s = lax.axis_index("s")
        wid = s * _NC + c
        pltpu.sync_copy(ztile_hbm, buf_v)
        pltpu.sync_copy(buf_v, acc_sh.at[pl.ds(s * TR, TR)])
        pltpu.sync_copy(src_hbm.at[wid], src_v)
        pltpu.sync_copy(dst_hbm.at[wid], dst_v)
        plsc.subcore_barrier()

        for b in range(_NBUF):
            pltpu.async_copy(tbl_hbm.at[src_v.at[b]], rows_v.at[b], gsem.at[b])

        def body(g, carry):
            for b in range(_NBUF):
                j = g * _NBUF + b
                pltpu.make_async_copy(
                    tbl_hbm.at[src_v.at[j]], rows_v.at[b], gsem.at[b]).wait()
                pltpu.async_copy(
                    rows_v.at[b], acc_sh.at[dst_v.at[j]], ssem.at[b], add=True)

                @pl.when(g < G - 1)
                def _():
                    pltpu.make_async_copy(
                        rows_v.at[b], acc_sh.at[dst_v.at[j]], ssem.at[b]).wait()
                    pltpu.async_copy(
                        tbl_hbm.at[src_v.at[j + _NBUF]], rows_v.at[b], gsem.at[b])
            return carry

        lax.fori_loop(0, G, body, 0)
        for b in range(_NBUF):
            j = KC - _NBUF + b
            pltpu.make_async_copy(
                rows_v.at[b], acc_sh.at[dst_v.at[j]], ssem.at[b]).wait()
        plsc.subcore_barrier()
        pltpu.sync_copy(acc_sh.at[pl.ds(s * TR, TR)], buf_v)
        pltpu.sync_copy(buf_v, out_hbm.at[c, pl.ds(s * TR, TR)])

    return prop_kernel


def _mm1_body(NP, feat_ref, mask_ref, w1_ref, deg_ref, dinv_ref, p1_ref):
    n = feat_ref.shape[0]
    deg = deg_ref[:n] + deg_ref[NP:NP + n]
    dinv = lax.rsqrt(deg + 1.0)
    dinv_ref[...] = dinv
    m = jnp.reshape(mask_ref[...] * dinv, (n, 1))
    p1_ref[...] = jnp.dot(feat_ref[...], w1_ref[...],
                          preferred_element_type=jnp.float32,
                          precision=lax.Precision.HIGHEST) * m


def _layer_body(parts_ref, p1_ref, dinv_ref, b1_ref, w2_ref, q_ref):
    n = p1_ref.shape[0]
    dcol = jnp.reshape(dinv_ref[...], (n, 1))
    agg = (parts_ref[0, :n] + parts_ref[1, :n] + p1_ref[...]) * dcol
    h1 = jnp.maximum(agg + b1_ref[...], 0.0)
    q_ref[...] = jnp.dot(h1, w2_ref[...], preferred_element_type=jnp.float32,
                         precision=lax.Precision.HIGHEST) * dcol


def _final_body(parts_ref, q_ref, dinv_ref, b2_ref, wd_ref, bd_ref, out_ref):
    n = q_ref.shape[0]
    dcol = jnp.reshape(dinv_ref[...], (n, 1))
    agg = (parts_ref[0, :n] + parts_ref[1, :n] + q_ref[...]) * dcol
    h2 = jnp.maximum(agg + b2_ref[...], 0.0)
    pooled = jnp.sum(h2, axis=0, keepdims=True)
    out_ref[...] = jnp.dot(pooled, wd_ref[...], preferred_element_type=jnp.float32,
                           precision=lax.Precision.HIGHEST) + bd_ref[...]


def kernel(x, edge_index, W1, b1, W2, b2, Wd, bd):
    n = x.shape[0]
    d = W1.shape[0]
    H = W1.shape[1]
    e = edge_index.shape[1]

    NP = -(-n // 128) * 128
    EPT = -(-e // _NW)
    KC = -(-(-(-EPT // _C)) // _NBUF) * _NBUF
    EPTP = KC * _C

    ei = edge_index.astype(jnp.int32)
    if _NW * EPT != e:
        pad = jnp.concatenate([jnp.zeros((1, _NW * EPT - e), jnp.int32),
                               jnp.full((1, _NW * EPT - e), n, jnp.int32)])
        ei = jnp.concatenate([ei, pad], axis=1)
    ei3 = ei.reshape(2, _NW, EPT)
    if EPTP != EPT:
        ntrash = NP - n
        j = jnp.arange(EPTP - EPT, dtype=jnp.int32)[None, :]
        w = jnp.arange(_NW, dtype=jnp.int32)[:, None]
        dpad = n + (j + w * 7) % ntrash
        padt = jnp.stack([jnp.zeros((_NW, EPTP - EPT), jnp.int32), dpad])
        ei3 = jnp.concatenate([ei3, padt], axis=2)
    src3 = ei3[0].reshape(_NW, KC, _C)
    dst3 = ei3[1].reshape(_NW, KC, _C)

    ones_c = jnp.ones((_C,), jnp.float32)
    zrow = jnp.zeros((NP // _NS,), jnp.float32)
    ztile = jnp.zeros((NP // _NS, H), jnp.float32)

    deg_kernel = _make_deg_kernel(NP, KC)
    prop_kernel = _make_prop_kernel(NP, KC, H)

    deg1d = deg_kernel(dst3, ones_c, zrow)
    feat = x[:, :d]
    mask = x[:, d]

    dinv, p1 = pl.pallas_call(
        functools.partial(_mm1_body, NP),
        out_shape=(jax.ShapeDtypeStruct((n,), jnp.float32),
                   jax.ShapeDtypeStruct((n, H), jnp.float32)),
    )(feat, mask, W1, deg1d)

    parts1 = prop_kernel(p1, src3, dst3, ztile)

    q = pl.pallas_call(
        _layer_body,
        out_shape=jax.ShapeDtypeStruct((n, H), jnp.float32),
    )(parts1, p1, dinv, b1, W2)

    parts2 = prop_kernel(q, src3, dst3, ztile)

    out = pl.pallas_call(
        _final_body,
        out_shape=jax.ShapeDtypeStruct((1, 1), jnp.float32),
    )(parts2, q, dinv, b2, Wd, bd.reshape(1, 1))

    return out
